# baseline (speedup 1.0000x reference)
import functools

import jax
import jax.numpy as jnp
from jax import lax
from jax.experimental import pallas as pl
from jax.experimental.pallas import tpu as pltpu
from jax.experimental.pallas import tpu_sc as plsc

NC = 2
NS = 16
NW = NC * NS

CHUNK = 128
DH = 64
NB = 4


def _sc_scatter_rows(feat_half, src_off, dst, n_pad):
    cpt = src_off.shape[2]
    rows_per_tile = n_pad // NS
    assert cpt % NB == 0 and cpt >= 2 * NB

    mesh = plsc.VectorSubcoreMesh(core_axis_name="c", subcore_axis_name="s")

    @functools.partial(
        pl.kernel,
        mesh=mesh,
        compiler_params=pltpu.CompilerParams(use_tc_tiling_on_sc=False),
        out_type=jax.ShapeDtypeStruct((NC, n_pad, DH), jnp.float32),
        scratch_types=[
            pltpu.VMEM((cpt, CHUNK), jnp.int32),
            pltpu.VMEM((cpt, CHUNK), jnp.int32),
            pltpu.VMEM((NB, CHUNK, DH), jnp.float32),
            pltpu.VMEM_SHARED((n_pad, DH), jnp.float32),
            [pltpu.SemaphoreType.DMA] * NB,
            pltpu.SemaphoreType.DMA,
        ],
    )
    def k(feat_hbm, src_hbm, dst_hbm, out_hbm, src_v, dst_v, rows_v,
          acc_sh, gsems, isem):
        c = lax.axis_index("c")
        s = lax.axis_index("s")

        icopy_s = pltpu.async_copy(src_hbm.at[c, s], src_v, isem)
        icopy_d = pltpu.async_copy(dst_hbm.at[s], dst_v, isem)

        zblk = jnp.zeros((16,), jnp.float32)
        for r in range(8):
            for l in range(DH // 16):
                rows_v[0, r, pl.ds(l * 16, 16)] = zblk
        row0 = s * rows_per_tile

        def zero_body(j, _):
            pltpu.sync_copy(rows_v.at[0, pl.ds(0, 8)],
                            acc_sh.at[pl.ds(row0 + j * 8, 8)])
            return 0

        lax.fori_loop(0, rows_per_tile // 8, zero_body, 0)
        icopy_s.wait()
        icopy_d.wait()
        plsc.subcore_barrier()

        def gather(j, b):
            pltpu.async_copy(feat_hbm.at[src_v.at[j]], rows_v.at[b],
                             gsems[b])

        def gwait(b):
            pltpu.make_async_copy(feat_hbm.at[pl.ds(0, CHUNK)],
                                  rows_v.at[b], gsems[b]).wait()

        def scatter(j, b):
            pltpu.sync_copy(rows_v.at[b], acc_sh.at[dst_v.at[j]], add=True)

        for b in range(NB):
            gather(b, b)

        def group(g, _):
            for b in range(NB):
                j = g * NB + b
                gwait(b)
                scatter(j, b)
                gather(j + NB, b)
            return 0

        lax.fori_loop(0, cpt // NB - 1, group, 0)
        for b in range(NB):
            j = cpt - NB + b
            gwait(b)
            scatter(j, b)
        plsc.subcore_barrier()

        pltpu.sync_copy(acc_sh.at[pl.ds(row0, rows_per_tile)],
                        out_hbm.at[c, pl.ds(row0, rows_per_tile)])

    return k(feat_half, src_off, dst)


def _hidden_kernel(p_ref, x_ref, w_ref, b_ref, o_ref):
    agg = jnp.concatenate([p_ref[0], p_ref[1]], axis=1) + x_ref[...]
    h = jnp.dot(agg, w_ref[...], preferred_element_type=jnp.float32)
    o_ref[...] = jnp.maximum(h + b_ref[...], 0.0)


def _heads_kernel(p_ref, h_ref, w2_ref, w3_ref, mu_ref, lv_ref):
    agg = jnp.concatenate([p_ref[0], p_ref[1]], axis=1) + h_ref[...]
    mu_ref[...] = jnp.dot(agg, w2_ref[...], preferred_element_type=jnp.float32)
    lv_ref[...] = jnp.dot(agg, w3_ref[...], preferred_element_type=jnp.float32)


def _adj_kernel(a_ref, b_ref, o_ref):
    o_ref[...] = lax.dot_general(
        a_ref[...], b_ref[...], (((1,), (1,)), ((), ())),
        preferred_element_type=jnp.float32)


def kernel(x, edge_index, W1, b1, W2, W3):
    n, d_in = x.shape
    e = edge_index.shape[1]
    h2 = W2.shape[1]

    src = edge_index[0].astype(jnp.int32)
    dst = edge_index[1].astype(jnp.int32)

    n_pad = ((n + NS * 8 - 1) // (NS * 8)) * (NS * 8)
    estep = NS * CHUNK * NB
    e_pad = ((e + estep - 1) // estep) * estep
    if e_pad != e:
        pad = e_pad - e
        src = jnp.concatenate([src, jnp.zeros((pad,), jnp.int32)])
        dst = jnp.concatenate([dst, jnp.full((pad,), n_pad - 1, jnp.int32)])
    cpt = e_pad // (NS * CHUNK)
    src_off = (2 * src)[None, :] + jnp.arange(NC, dtype=jnp.int32)[:, None]
    src_off = src_off.reshape(NC, NS, cpt, CHUNK)
    dst = dst.reshape(NS, cpt, CHUNK)

    parts1 = _sc_scatter_rows(x.reshape(NC * n, DH), src_off, dst, n_pad)

    rb = 1000
    grid = (n // rb,)
    hidden1 = pl.pallas_call(
        _hidden_kernel,
        grid=grid,
        in_specs=[
            pl.BlockSpec((NC, rb, DH), lambda i: (0, i, 0)),
            pl.BlockSpec((rb, d_in), lambda i: (i, 0)),
            pl.BlockSpec((d_in, d_in), lambda i: (0, 0)),
            pl.BlockSpec((d_in,), lambda i: (0,)),
        ],
        out_specs=pl.BlockSpec((rb, d_in), lambda i: (i, 0)),
        out_shape=jax.ShapeDtypeStruct((n, d_in), jnp.float32),
    )(parts1, x, W1, b1)

    parts2 = _sc_scatter_rows(hidden1.reshape(NC * n, DH), src_off, dst,
                              n_pad)

    mu, logvar = pl.pallas_call(
        _heads_kernel,
        grid=grid,
        in_specs=[
            pl.BlockSpec((NC, rb, DH), lambda i: (0, i, 0)),
            pl.BlockSpec((rb, d_in), lambda i: (i, 0)),
            pl.BlockSpec((d_in, h2), lambda i: (0, 0)),
            pl.BlockSpec((d_in, h2), lambda i: (0, 0)),
        ],
        out_specs=[
            pl.BlockSpec((rb, h2), lambda i: (i, 0)),
            pl.BlockSpec((rb, h2), lambda i: (i, 0)),
        ],
        out_shape=[
            jax.ShapeDtypeStruct((n, h2), jnp.float32),
            jax.ShapeDtypeStruct((n, h2), jnp.float32),
        ],
    )(parts2, hidden1, W2, W3)

    arb, acb = 512, 2048
    gi = (n + arb - 1) // arb
    gj = (n + acb - 1) // acb
    adj = pl.pallas_call(
        _adj_kernel,
        grid=(gi, gj),
        in_specs=[
            pl.BlockSpec((arb, h2), lambda i, j: (i, 0)),
            pl.BlockSpec((acb, h2), lambda i, j: (j, 0)),
        ],
        out_specs=pl.BlockSpec((arb, acb), lambda i, j: (i, j)),
        out_shape=jax.ShapeDtypeStruct((n, n), jnp.float32),
    )(mu, mu)

    return (adj, mu, logvar)


# speedup vs baseline: 1.1482x; 1.1482x over previous
import functools

import jax
import jax.numpy as jnp
from jax import lax
from jax.experimental import pallas as pl
from jax.experimental.pallas import tpu as pltpu
from jax.experimental.pallas import tpu_sc as plsc

NC = 2
NS = 16
NW = NC * NS

CHUNK = 128
DH = 64
NB = 4


def _sc_scatter_rows(feat_half, src_off, dst, n_pad):
    cpt = src_off.shape[2]
    rows_per_tile = n_pad // NS
    assert cpt % NB == 0 and cpt >= 2 * NB

    mesh = plsc.VectorSubcoreMesh(core_axis_name="c", subcore_axis_name="s")

    @functools.partial(
        pl.kernel,
        mesh=mesh,
        compiler_params=pltpu.CompilerParams(use_tc_tiling_on_sc=False),
        out_type=jax.ShapeDtypeStruct((NC, n_pad, DH), jnp.float32),
        scratch_types=[
            pltpu.VMEM((cpt, CHUNK), jnp.int32),
            pltpu.VMEM((cpt, CHUNK), jnp.int32),
            pltpu.VMEM((NB, CHUNK, DH), jnp.float32),
            pltpu.VMEM_SHARED((n_pad, DH), jnp.float32),
            [pltpu.SemaphoreType.DMA] * NB,
            pltpu.SemaphoreType.DMA,
        ],
    )
    def k(feat_hbm, src_hbm, dst_hbm, out_hbm, src_v, dst_v, rows_v,
          acc_sh, gsems, isem):
        c = lax.axis_index("c")
        s = lax.axis_index("s")

        icopy_s = pltpu.async_copy(src_hbm.at[c, s], src_v, isem)
        icopy_d = pltpu.async_copy(dst_hbm.at[s], dst_v, isem)

        zblk = jnp.zeros((16,), jnp.float32)
        for r in range(8):
            for l in range(DH // 16):
                rows_v[0, r, pl.ds(l * 16, 16)] = zblk
        row0 = s * rows_per_tile

        def zero_body(j, _):
            pltpu.sync_copy(rows_v.at[0, pl.ds(0, 8)],
                            acc_sh.at[pl.ds(row0 + j * 8, 8)])
            return 0

        lax.fori_loop(0, rows_per_tile // 8, zero_body, 0)
        icopy_s.wait()
        icopy_d.wait()
        plsc.subcore_barrier()

        def gather(j, b):
            pltpu.async_copy(feat_hbm.at[src_v.at[j]], rows_v.at[b],
                             gsems[b])

        def gwait(b):
            pltpu.make_async_copy(feat_hbm.at[pl.ds(0, CHUNK)],
                                  rows_v.at[b], gsems[b]).wait()

        def scatter(j, b):
            pltpu.sync_copy(rows_v.at[b], acc_sh.at[dst_v.at[j]], add=True)

        for b in range(NB):
            gather(b, b)

        def group(g, _):
            for b in range(NB):
                j = g * NB + b
                gwait(b)
                scatter(j, b)
                gather(j + NB, b)
            return 0

        lax.fori_loop(0, cpt // NB - 1, group, 0)
        for b in range(NB):
            j = cpt - NB + b
            gwait(b)
            scatter(j, b)
        plsc.subcore_barrier()

        pltpu.sync_copy(acc_sh.at[pl.ds(row0, rows_per_tile)],
                        out_hbm.at[c, pl.ds(row0, rows_per_tile)])

    return k(feat_half, src_off, dst)


def _hidden_kernel(p_ref, x_ref, w_ref, b_ref, o_ref):
    agg = jnp.concatenate([p_ref[0], p_ref[1]], axis=1) + x_ref[...]
    h = jnp.dot(agg, w_ref[...], preferred_element_type=jnp.float32)
    h = jnp.maximum(h + b_ref[...], 0.0)
    o_ref[0] = h[:, :DH]
    o_ref[1] = h[:, DH:]


def _heads_kernel(p_ref, h_ref, w2_ref, w3_ref, mu_ref, lv_ref):
    agg = (jnp.concatenate([p_ref[0], p_ref[1]], axis=1)
           + jnp.concatenate([h_ref[0], h_ref[1]], axis=1))
    mu_ref[...] = jnp.dot(agg, w2_ref[...], preferred_element_type=jnp.float32)
    lv_ref[...] = jnp.dot(agg, w3_ref[...], preferred_element_type=jnp.float32)


def _adj_kernel(a_ref, b_ref, o_ref):
    o_ref[...] = lax.dot_general(
        a_ref[...], b_ref[...], (((1,), (1,)), ((), ())),
        preferred_element_type=jnp.float32)


def kernel(x, edge_index, W1, b1, W2, W3):
    n, d_in = x.shape
    e = edge_index.shape[1]
    h2 = W2.shape[1]

    src = edge_index[0].astype(jnp.int32)
    dst = edge_index[1].astype(jnp.int32)

    n_pad = ((n + NS * 8 - 1) // (NS * 8)) * (NS * 8)
    estep = NS * CHUNK * NB
    e_pad = ((e + estep - 1) // estep) * estep
    if e_pad != e:
        pad = e_pad - e
        src = jnp.concatenate([src, jnp.zeros((pad,), jnp.int32)])
        dst = jnp.concatenate([dst, jnp.full((pad,), n_pad - 1, jnp.int32)])
    cpt = e_pad // (NS * CHUNK)
    src_off = src[None, :] + (jnp.arange(NC, dtype=jnp.int32) * n)[:, None]
    src_off = src_off.reshape(NC, NS, cpt, CHUNK)
    dst = dst.reshape(NS, cpt, CHUNK)

    x_pair = jnp.concatenate([x[:, :DH], x[:, DH:]], axis=0)
    parts1 = _sc_scatter_rows(x_pair, src_off, dst, n_pad)

    rb = 1000
    grid = (n // rb,)
    hidden1 = pl.pallas_call(
        _hidden_kernel,
        grid=grid,
        in_specs=[
            pl.BlockSpec((NC, rb, DH), lambda i: (0, i, 0)),
            pl.BlockSpec((rb, d_in), lambda i: (i, 0)),
            pl.BlockSpec((d_in, d_in), lambda i: (0, 0)),
            pl.BlockSpec((d_in,), lambda i: (0,)),
        ],
        out_specs=pl.BlockSpec((NC, rb, DH), lambda i: (0, i, 0)),
        out_shape=jax.ShapeDtypeStruct((NC, n, DH), jnp.float32),
    )(parts1, x, W1, b1)

    parts2 = _sc_scatter_rows(hidden1.reshape(NC * n, DH), src_off, dst,
                              n_pad)

    mu, logvar = pl.pallas_call(
        _heads_kernel,
        grid=grid,
        in_specs=[
            pl.BlockSpec((NC, rb, DH), lambda i: (0, i, 0)),
            pl.BlockSpec((NC, rb, DH), lambda i: (0, i, 0)),
            pl.BlockSpec((d_in, h2), lambda i: (0, 0)),
            pl.BlockSpec((d_in, h2), lambda i: (0, 0)),
        ],
        out_specs=[
            pl.BlockSpec((rb, h2), lambda i: (i, 0)),
            pl.BlockSpec((rb, h2), lambda i: (i, 0)),
        ],
        out_shape=[
            jax.ShapeDtypeStruct((n, h2), jnp.float32),
            jax.ShapeDtypeStruct((n, h2), jnp.float32),
        ],
    )(parts2, hidden1, W2, W3)

    arb, acb = 512, 2048
    gi = (n + arb - 1) // arb
    gj = (n + acb - 1) // acb
    adj = pl.pallas_call(
        _adj_kernel,
        grid=(gi, gj),
        in_specs=[
            pl.BlockSpec((arb, h2), lambda i, j: (i, 0)),
            pl.BlockSpec((acb, h2), lambda i, j: (j, 0)),
        ],
        out_specs=pl.BlockSpec((arb, acb), lambda i, j: (i, j)),
        out_shape=jax.ShapeDtypeStruct((n, n), jnp.float32),
    )(mu, mu)

    return (adj, mu, logvar)


# speedup vs baseline: 1.8019x; 1.5693x over previous
import functools

import jax
import jax.numpy as jnp
from jax import lax
from jax.experimental import pallas as pl
from jax.experimental.pallas import tpu as pltpu
from jax.experimental.pallas import tpu_sc as plsc

NC = 2
NS = 16
NW = NC * NS

CHUNK = 128
DH = 64
NB = 4
G = 8


def _sc_scatter_rows(feat, src, dst, n_pad):
    ngroups = src.shape[1]
    rows_per_tile = n_pad // NS
    assert ngroups % 2 == 0 and ngroups >= 4

    mesh = plsc.VectorSubcoreMesh(core_axis_name="c", subcore_axis_name="s")

    @functools.partial(
        pl.kernel,
        mesh=mesh,
        compiler_params=pltpu.CompilerParams(use_tc_tiling_on_sc=False),
        out_type=jax.ShapeDtypeStruct((NC, n_pad, DH), jnp.float32),
        scratch_types=[
            pltpu.VMEM((2, G, CHUNK), jnp.int32),
            pltpu.VMEM((2, G, CHUNK), jnp.int32),
            pltpu.VMEM((NB, CHUNK, DH), jnp.float32),
            pltpu.VMEM_SHARED((n_pad, DH), jnp.float32),
            pltpu.VMEM_SHARED((n_pad, DH), jnp.float32),
            [pltpu.SemaphoreType.DMA] * NB,
            [pltpu.SemaphoreType.DMA] * 2,
        ],
    )
    def k(feat_hbm, src_hbm, dst_hbm, out_hbm, srcg, dstg, rows_v,
          tab_sh, acc_sh, gsems, isems):
        c = lax.axis_index("c")
        s = lax.axis_index("s")
        row0 = s * rows_per_tile

        def stage_idx(g, p):
            pltpu.async_copy(src_hbm.at[s, g], srcg.at[p], isems[p])
            pltpu.async_copy(dst_hbm.at[s, g], dstg.at[p], isems[p])

        def idx_wait(p):
            for _ in range(2):
                pltpu.make_async_copy(src_hbm.at[s, 0], srcg.at[p],
                                      isems[p]).wait()

        stage_idx(0, 0)
        pltpu.sync_copy(
            feat_hbm.at[pl.ds(row0, rows_per_tile), pl.ds(c * DH, DH)],
            tab_sh.at[pl.ds(row0, rows_per_tile)])

        zblk = jnp.zeros((16,), jnp.float32)
        for r in range(8):
            for l in range(DH // 16):
                rows_v[0, r, pl.ds(l * 16, 16)] = zblk

        def zero_body(j, _):
            pltpu.sync_copy(rows_v.at[0, pl.ds(0, 8)],
                            acc_sh.at[pl.ds(row0 + j * 8, 8)])
            return 0

        lax.fori_loop(0, rows_per_tile // 8, zero_body, 0)
        idx_wait(0)
        plsc.subcore_barrier()

        def gather(p, jj, b):
            pltpu.async_copy(tab_sh.at[srcg.at[p, jj]], rows_v.at[b],
                             gsems[b])

        def gwait(b):
            pltpu.make_async_copy(feat_hbm.at[pl.ds(0, CHUNK), pl.ds(0, DH)],
                                  rows_v.at[b], gsems[b]).wait()

        def scatter(p, t, b):
            pltpu.sync_copy(rows_v.at[b], acc_sh.at[dstg.at[p, t]],
                            add=True)

        def run_group(g, p, stage_next, gather_next):
            if stage_next:
                stage_idx(g + 1, 1 - p)
            for t in range(G):
                b = t % NB
                if t == G - NB and gather_next:
                    idx_wait(1 - p)
                gwait(b)
                scatter(p, t, b)
                if t < G - NB:
                    gather(p, t + NB, b)
                elif gather_next:
                    gather(1 - p, t + NB - G, b)
            return

        for b in range(NB):
            gather(0, b, b)
        run_group(0, 0, stage_next=True, gather_next=True)

        def pair_body(i, _):
            ga = 1 + 2 * i
            run_group(ga, 1, stage_next=True, gather_next=True)
            run_group(ga + 1, 0, stage_next=True, gather_next=True)
            return 0

        lax.fori_loop(0, (ngroups - 2) // 2, pair_body, 0)

        run_group(ngroups - 1, 1, stage_next=False, gather_next=False)
        plsc.subcore_barrier()

        pltpu.sync_copy(acc_sh.at[pl.ds(row0, rows_per_tile)],
                        out_hbm.at[c, pl.ds(row0, rows_per_tile)])

    return k(feat, src, dst)


def _hidden_kernel(p_ref, x_ref, w_ref, b_ref, o_ref):
    agg = jnp.concatenate([p_ref[0], p_ref[1]], axis=1) + x_ref[...]
    h = jnp.dot(agg, w_ref[...], preferred_element_type=jnp.float32)
    o_ref[...] = jnp.maximum(h + b_ref[...], 0.0)


def _heads_kernel(p_ref, h_ref, w2_ref, w3_ref, mu_ref, lv_ref):
    agg = jnp.concatenate([p_ref[0], p_ref[1]], axis=1) + h_ref[...]
    mu_ref[...] = jnp.dot(agg, w2_ref[...], preferred_element_type=jnp.float32)
    lv_ref[...] = jnp.dot(agg, w3_ref[...], preferred_element_type=jnp.float32)


def _adj_kernel(a_ref, b_ref, o_ref):
    o_ref[...] = lax.dot_general(
        a_ref[...], b_ref[...], (((1,), (1,)), ((), ())),
        preferred_element_type=jnp.float32)


def kernel(x, edge_index, W1, b1, W2, W3):
    n, d_in = x.shape
    e = edge_index.shape[1]
    h2 = W2.shape[1]

    src = edge_index[0].astype(jnp.int32)
    dst = edge_index[1].astype(jnp.int32)

    n_pad = ((n + NS * 8 - 1) // (NS * 8)) * (NS * 8)
    estep = NS * CHUNK * G * 2
    e_pad = ((e + estep - 1) // estep) * estep
    if e_pad != e:
        pad = e_pad - e
        src = jnp.concatenate([src, jnp.zeros((pad,), jnp.int32)])
        dst = jnp.concatenate([dst, jnp.full((pad,), n_pad - 1, jnp.int32)])
    ngroups = e_pad // (NS * CHUNK * G)
    src = src.reshape(NS, ngroups, G, CHUNK)
    dst = dst.reshape(NS, ngroups, G, CHUNK)

    x_padded = jnp.concatenate(
        [x, jnp.zeros((n_pad - n, d_in), jnp.float32)], axis=0)

    parts1 = _sc_scatter_rows(x_padded, src, dst, n_pad)

    rb = 1000
    grid = (n // rb,)
    hidden1 = pl.pallas_call(
        _hidden_kernel,
        grid=grid,
        in_specs=[
            pl.BlockSpec((NC, rb, DH), lambda i: (0, i, 0)),
            pl.BlockSpec((rb, d_in), lambda i: (i, 0)),
            pl.BlockSpec((d_in, d_in), lambda i: (0, 0)),
            pl.BlockSpec((d_in,), lambda i: (0,)),
        ],
        out_specs=pl.BlockSpec((rb, d_in), lambda i: (i, 0)),
        out_shape=jax.ShapeDtypeStruct((n_pad, d_in), jnp.float32),
    )(parts1, x, W1, b1)

    parts2 = _sc_scatter_rows(hidden1, src, dst, n_pad)

    mu, logvar = pl.pallas_call(
        _heads_kernel,
        grid=grid,
        in_specs=[
            pl.BlockSpec((NC, rb, DH), lambda i: (0, i, 0)),
            pl.BlockSpec((rb, d_in), lambda i: (i, 0)),
            pl.BlockSpec((d_in, h2), lambda i: (0, 0)),
            pl.BlockSpec((d_in, h2), lambda i: (0, 0)),
        ],
        out_specs=[
            pl.BlockSpec((rb, h2), lambda i: (i, 0)),
            pl.BlockSpec((rb, h2), lambda i: (i, 0)),
        ],
        out_shape=[
            jax.ShapeDtypeStruct((n, h2), jnp.float32),
            jax.ShapeDtypeStruct((n, h2), jnp.float32),
        ],
    )(parts2, hidden1, W2, W3)

    arb, acb = 512, 2048
    gi = (n + arb - 1) // arb
    gj = (n + acb - 1) // acb
    adj = pl.pallas_call(
        _adj_kernel,
        grid=(gi, gj),
        in_specs=[
            pl.BlockSpec((arb, h2), lambda i, j: (i, 0)),
            pl.BlockSpec((acb, h2), lambda i, j: (j, 0)),
        ],
        out_specs=pl.BlockSpec((arb, acb), lambda i, j: (i, j)),
        out_shape=jax.ShapeDtypeStruct((n, n), jnp.float32),
    )(mu, mu)

    return (adj, mu, logvar)


# speedup vs baseline: 1.9342x; 1.0734x over previous
import functools

import jax
import jax.numpy as jnp
from jax import lax
from jax.experimental import pallas as pl
from jax.experimental.pallas import tpu as pltpu
from jax.experimental.pallas import tpu_sc as plsc

NC = 2
NS = 16
NW = NC * NS

CHUNK = 128
DH = 64
NB = 4
G = 8


def _sc_scatter_rows(feat, src, dst, n_pad):
    ngroups = src.shape[1]
    rows_per_tile = n_pad // NS
    assert ngroups % 2 == 0 and ngroups >= 4

    mesh = plsc.VectorSubcoreMesh(core_axis_name="c", subcore_axis_name="s")

    @functools.partial(
        pl.kernel,
        mesh=mesh,
        compiler_params=pltpu.CompilerParams(use_tc_tiling_on_sc=False),
        out_type=jax.ShapeDtypeStruct((NC, n_pad, DH), jnp.float32),
        scratch_types=[
            pltpu.VMEM((2, G, CHUNK), jnp.int32),
            pltpu.VMEM((2, G, CHUNK), jnp.int32),
            pltpu.VMEM((NB, CHUNK, DH), jnp.float32),
            pltpu.VMEM_SHARED((n_pad, DH), jnp.float32),
            pltpu.VMEM_SHARED((n_pad, DH), jnp.float32),
            [pltpu.SemaphoreType.DMA] * NB,
            [pltpu.SemaphoreType.DMA] * 2,
        ],
    )
    def k(feat_hbm, src_hbm, dst_hbm, out_hbm, srcg, dstg, rows_v,
          tab_sh, acc_sh, gsems, isems):
        c = lax.axis_index("c")
        s = lax.axis_index("s")
        row0 = s * rows_per_tile

        def stage_idx(g, p):
            pltpu.async_copy(src_hbm.at[s, g], srcg.at[p], isems[p])
            pltpu.async_copy(dst_hbm.at[s, g], dstg.at[p], isems[p])

        def idx_wait(p):
            for _ in range(2):
                pltpu.make_async_copy(src_hbm.at[s, 0], srcg.at[p],
                                      isems[p]).wait()

        stage_idx(0, 0)
        pltpu.sync_copy(
            feat_hbm.at[pl.ds(row0, rows_per_tile), pl.ds(c * DH, DH)],
            tab_sh.at[pl.ds(row0, rows_per_tile)])

        zblk = jnp.zeros((16,), jnp.float32)
        for r in range(8):
            for l in range(DH // 16):
                rows_v[0, r, pl.ds(l * 16, 16)] = zblk

        def zero_body(j, _):
            pltpu.sync_copy(rows_v.at[0, pl.ds(0, 8)],
                            acc_sh.at[pl.ds(row0 + j * 8, 8)])
            return 0

        lax.fori_loop(0, rows_per_tile // 8, zero_body, 0)
        idx_wait(0)
        plsc.subcore_barrier()

        def gather(p, jj, b):
            pltpu.async_copy(tab_sh.at[srcg.at[p, jj]], rows_v.at[b],
                             gsems[b])

        def gwait(b):
            pltpu.make_async_copy(feat_hbm.at[pl.ds(0, CHUNK), pl.ds(0, DH)],
                                  rows_v.at[b], gsems[b]).wait()

        def scatter(p, t, b):
            pltpu.sync_copy(rows_v.at[b], acc_sh.at[dstg.at[p, t]],
                            add=True)

        def run_group(g, p, stage_next, gather_next):
            if stage_next:
                stage_idx(g + 1, 1 - p)
            for t in range(G):
                b = t % NB
                if t == G - NB and gather_next:
                    idx_wait(1 - p)
                gwait(b)
                scatter(p, t, b)
                if t < G - NB:
                    gather(p, t + NB, b)
                elif gather_next:
                    gather(1 - p, t + NB - G, b)
            return

        for b in range(NB):
            gather(0, b, b)
        run_group(0, 0, stage_next=True, gather_next=True)

        def pair_body(i, _):
            ga = 1 + 2 * i
            run_group(ga, 1, stage_next=True, gather_next=True)
            run_group(ga + 1, 0, stage_next=True, gather_next=True)
            return 0

        lax.fori_loop(0, (ngroups - 2) // 2, pair_body, 0)

        run_group(ngroups - 1, 1, stage_next=False, gather_next=False)
        plsc.subcore_barrier()

        pltpu.sync_copy(acc_sh.at[pl.ds(row0, rows_per_tile)],
                        out_hbm.at[c, pl.ds(row0, rows_per_tile)])

    return k(feat, src, dst)


def _hidden_kernel(p_ref, x_ref, w_ref, b_ref, o_ref):
    agg = jnp.concatenate([p_ref[0], p_ref[1]], axis=1) + x_ref[...]
    h = jnp.dot(agg, w_ref[...], preferred_element_type=jnp.float32)
    o_ref[...] = jnp.maximum(h + b_ref[...], 0.0)


def _heads_kernel(p_ref, h_ref, w2_ref, w3_ref, mu_ref, lv_ref):
    agg = jnp.concatenate([p_ref[0], p_ref[1]], axis=1) + h_ref[...]
    mu_ref[...] = jnp.dot(agg, w2_ref[...], preferred_element_type=jnp.float32)
    lv_ref[...] = jnp.dot(agg, w3_ref[...], preferred_element_type=jnp.float32)


def _adj_kernel(a_ref, b_ref, o_ref):
    o_ref[...] = lax.dot_general(
        a_ref[...], b_ref[...], (((1,), (1,)), ((), ())),
        preferred_element_type=jnp.float32)


def kernel(x, edge_index, W1, b1, W2, W3):
    n, d_in = x.shape
    e = edge_index.shape[1]
    h2 = W2.shape[1]

    src = edge_index[0].astype(jnp.int32)
    dst = edge_index[1].astype(jnp.int32)

    n_pad = ((n + NS * 8 - 1) // (NS * 8)) * (NS * 8)
    estep = NS * CHUNK * G * 2
    e_pad = ((e + estep - 1) // estep) * estep
    if e_pad != e:
        pad = e_pad - e
        src = jnp.concatenate([src, jnp.zeros((pad,), jnp.int32)])
        dst = jnp.concatenate([dst, jnp.full((pad,), n_pad - 1, jnp.int32)])
    ngroups = e_pad // (NS * CHUNK * G)
    src = src.reshape(NS, ngroups, G, CHUNK)
    dst = dst.reshape(NS, ngroups, G, CHUNK)

    x_padded = jnp.concatenate(
        [x, jnp.zeros((n_pad - n, d_in), jnp.float32)], axis=0)

    parts1 = _sc_scatter_rows(x_padded, src, dst, n_pad)

    rb = 1000
    grid = (n // rb,)
    hidden1 = pl.pallas_call(
        _hidden_kernel,
        grid=grid,
        in_specs=[
            pl.BlockSpec((NC, rb, DH), lambda i: (0, i, 0)),
            pl.BlockSpec((rb, d_in), lambda i: (i, 0)),
            pl.BlockSpec((d_in, d_in), lambda i: (0, 0)),
            pl.BlockSpec((d_in,), lambda i: (0,)),
        ],
        out_specs=pl.BlockSpec((rb, d_in), lambda i: (i, 0)),
        out_shape=jax.ShapeDtypeStruct((n_pad, d_in), jnp.float32),
    )(parts1, x, W1, b1)

    parts2 = _sc_scatter_rows(hidden1, src, dst, n_pad)

    mu, logvar = pl.pallas_call(
        _heads_kernel,
        grid=grid,
        in_specs=[
            pl.BlockSpec((NC, rb, DH), lambda i: (0, i, 0)),
            pl.BlockSpec((rb, d_in), lambda i: (i, 0)),
            pl.BlockSpec((d_in, h2), lambda i: (0, 0)),
            pl.BlockSpec((d_in, h2), lambda i: (0, 0)),
        ],
        out_specs=[
            pl.BlockSpec((rb, h2), lambda i: (i, 0)),
            pl.BlockSpec((rb, h2), lambda i: (i, 0)),
        ],
        out_shape=[
            jax.ShapeDtypeStruct((n, h2), jnp.float32),
            jax.ShapeDtypeStruct((n, h2), jnp.float32),
        ],
    )(parts2, hidden1, W2, W3)

    arb, acb = 1024, 2048
    gi = (n + arb - 1) // arb
    gj = (n + acb - 1) // acb
    adj = pl.pallas_call(
        _adj_kernel,
        grid=(gi, gj),
        in_specs=[
            pl.BlockSpec((arb, h2), lambda i, j: (i, 0)),
            pl.BlockSpec((acb, h2), lambda i, j: (j, 0)),
        ],
        out_specs=pl.BlockSpec((arb, acb), lambda i, j: (i, j)),
        out_shape=jax.ShapeDtypeStruct((n, n), jnp.float32),
    )(mu, mu)

    return (adj, mu, logvar)


# speedup vs baseline: 1.9498x; 1.0081x over previous
import functools

import jax
import jax.numpy as jnp
from jax import lax
from jax.experimental import pallas as pl
from jax.experimental.pallas import tpu as pltpu
from jax.experimental.pallas import tpu_sc as plsc

NC = 2
NS = 16
NW = NC * NS

CHUNK = 128
DH = 64
NB = 4
G = 8


def _sc_scatter_rows(feat, src, dst, n_pad):
    ngroups = src.shape[1]
    rows_per_tile = n_pad // NS
    assert ngroups % 2 == 0 and ngroups >= 4

    mesh = plsc.VectorSubcoreMesh(core_axis_name="c", subcore_axis_name="s")

    @functools.partial(
        pl.kernel,
        mesh=mesh,
        compiler_params=pltpu.CompilerParams(use_tc_tiling_on_sc=False),
        out_type=jax.ShapeDtypeStruct((NC, n_pad, DH), jnp.float32),
        scratch_types=[
            pltpu.VMEM((2, G, CHUNK), jnp.int32),
            pltpu.VMEM((2, G, CHUNK), jnp.int32),
            pltpu.VMEM((NB, CHUNK, DH), jnp.float32),
            pltpu.VMEM_SHARED((n_pad, DH), jnp.float32),
            pltpu.VMEM_SHARED((n_pad, DH), jnp.float32),
            [pltpu.SemaphoreType.DMA] * NB,
            [pltpu.SemaphoreType.DMA] * 2,
        ],
    )
    def k(feat_hbm, src_hbm, dst_hbm, out_hbm, srcg, dstg, rows_v,
          tab_sh, acc_sh, gsems, isems):
        c = lax.axis_index("c")
        s = lax.axis_index("s")
        row0 = s * rows_per_tile

        def stage_idx(g, p):
            pltpu.async_copy(src_hbm.at[s, g], srcg.at[p], isems[p])
            pltpu.async_copy(dst_hbm.at[s, g], dstg.at[p], isems[p])

        def idx_wait(p):
            for _ in range(2):
                pltpu.make_async_copy(src_hbm.at[s, 0], srcg.at[p],
                                      isems[p]).wait()

        stage_idx(0, 0)
        pltpu.sync_copy(
            feat_hbm.at[pl.ds(row0, rows_per_tile), pl.ds(c * DH, DH)],
            tab_sh.at[pl.ds(row0, rows_per_tile)])

        zblk = jnp.zeros((16,), jnp.float32)
        for r in range(8):
            for l in range(DH // 16):
                rows_v[0, r, pl.ds(l * 16, 16)] = zblk

        def zero_body(j, _):
            pltpu.sync_copy(rows_v.at[0, pl.ds(0, 8)],
                            acc_sh.at[pl.ds(row0 + j * 8, 8)])
            return 0

        lax.fori_loop(0, rows_per_tile // 8, zero_body, 0)
        idx_wait(0)
        plsc.subcore_barrier()

        def gather(p, jj, b):
            pltpu.async_copy(tab_sh.at[srcg.at[p, jj]], rows_v.at[b],
                             gsems[b])

        def gwait(b):
            pltpu.make_async_copy(feat_hbm.at[pl.ds(0, CHUNK), pl.ds(0, DH)],
                                  rows_v.at[b], gsems[b]).wait()

        def scatter(p, t, b):
            pltpu.sync_copy(rows_v.at[b], acc_sh.at[dstg.at[p, t]],
                            add=True)

        def run_group(g, p, stage_next, gather_next):
            if stage_next:
                stage_idx(g + 1, 1 - p)
            for t in range(G):
                b = t % NB
                if t == G - NB and gather_next:
                    idx_wait(1 - p)
                gwait(b)
                scatter(p, t, b)
                if t < G - NB:
                    gather(p, t + NB, b)
                elif gather_next:
                    gather(1 - p, t + NB - G, b)
            return

        for b in range(NB):
            gather(0, b, b)
        run_group(0, 0, stage_next=True, gather_next=True)

        def pair_body(i, _):
            ga = 1 + 2 * i
            run_group(ga, 1, stage_next=True, gather_next=True)
            run_group(ga + 1, 0, stage_next=True, gather_next=True)
            return 0

        lax.fori_loop(0, (ngroups - 2) // 2, pair_body, 0)

        run_group(ngroups - 1, 1, stage_next=False, gather_next=False)
        plsc.subcore_barrier()

        pltpu.sync_copy(acc_sh.at[pl.ds(row0, rows_per_tile)],
                        out_hbm.at[c, pl.ds(row0, rows_per_tile)])

    return k(feat, src, dst)


def _hidden_kernel(p_ref, x_ref, w_ref, b_ref, o_ref):
    agg = jnp.concatenate([p_ref[0], p_ref[1]], axis=1) + x_ref[...]
    h = jnp.dot(agg, w_ref[...], preferred_element_type=jnp.float32)
    o_ref[...] = jnp.maximum(h + b_ref[...], 0.0)


def _heads_kernel(p_ref, h_ref, w2_ref, w3_ref, mu_ref, lv_ref):
    agg = jnp.concatenate([p_ref[0], p_ref[1]], axis=1) + h_ref[...]
    mu_ref[...] = jnp.dot(agg, w2_ref[...], preferred_element_type=jnp.float32)
    lv_ref[...] = jnp.dot(agg, w3_ref[...], preferred_element_type=jnp.float32)


def _adj_kernel(a_ref, b_ref, o_ref):
    o_ref[...] = lax.dot_general(
        a_ref[...], b_ref[...], (((1,), (1,)), ((), ())),
        preferred_element_type=jnp.float32)


def kernel(x, edge_index, W1, b1, W2, W3):
    n, d_in = x.shape
    e = edge_index.shape[1]
    h2 = W2.shape[1]

    src = edge_index[0].astype(jnp.int32)
    dst = edge_index[1].astype(jnp.int32)

    n_pad = ((n + NS * 8 - 1) // (NS * 8)) * (NS * 8)
    estep = NS * CHUNK * G * 2
    e_pad = ((e + estep - 1) // estep) * estep
    if e_pad != e:
        pad = e_pad - e
        src = jnp.concatenate([src, jnp.zeros((pad,), jnp.int32)])
        dst = jnp.concatenate([dst, jnp.full((pad,), n_pad - 1, jnp.int32)])
    ngroups = e_pad // (NS * CHUNK * G)
    src = src.reshape(NS, ngroups, G, CHUNK)
    dst = dst.reshape(NS, ngroups, G, CHUNK)

    x_padded = jnp.concatenate(
        [x, jnp.zeros((n_pad - n, d_in), jnp.float32)], axis=0)

    parts1 = _sc_scatter_rows(x_padded, src, dst, n_pad)

    rb = 1000
    grid = (n // rb,)
    hidden1 = pl.pallas_call(
        _hidden_kernel,
        grid=grid,
        in_specs=[
            pl.BlockSpec((NC, rb, DH), lambda i: (0, i, 0)),
            pl.BlockSpec((rb, d_in), lambda i: (i, 0)),
            pl.BlockSpec((d_in, d_in), lambda i: (0, 0)),
            pl.BlockSpec((d_in,), lambda i: (0,)),
        ],
        out_specs=pl.BlockSpec((rb, d_in), lambda i: (i, 0)),
        out_shape=jax.ShapeDtypeStruct((n_pad, d_in), jnp.float32),
    )(parts1, x, W1, b1)

    parts2 = _sc_scatter_rows(hidden1, src, dst, n_pad)

    mu, logvar = pl.pallas_call(
        _heads_kernel,
        grid=grid,
        in_specs=[
            pl.BlockSpec((NC, rb, DH), lambda i: (0, i, 0)),
            pl.BlockSpec((rb, d_in), lambda i: (i, 0)),
            pl.BlockSpec((d_in, h2), lambda i: (0, 0)),
            pl.BlockSpec((d_in, h2), lambda i: (0, 0)),
        ],
        out_specs=[
            pl.BlockSpec((rb, h2), lambda i: (i, 0)),
            pl.BlockSpec((rb, h2), lambda i: (i, 0)),
        ],
        out_shape=[
            jax.ShapeDtypeStruct((n, h2), jnp.float32),
            jax.ShapeDtypeStruct((n, h2), jnp.float32),
        ],
    )(parts2, hidden1, W2, W3)

    arb, acb = 1024, 5120
    gi = (n + arb - 1) // arb
    gj = (n + acb - 1) // acb
    adj = pl.pallas_call(
        _adj_kernel,
        grid=(gi, gj),
        in_specs=[
            pl.BlockSpec((arb, h2), lambda i, j: (i, 0)),
            pl.BlockSpec((acb, h2), lambda i, j: (j, 0)),
        ],
        out_specs=pl.BlockSpec((arb, acb), lambda i, j: (i, j)),
        out_shape=jax.ShapeDtypeStruct((n, n), jnp.float32),
    )(mu, mu)

    return (adj, mu, logvar)


# speedup vs baseline: 1.9825x; 1.0168x over previous
import functools

import jax
import jax.numpy as jnp
from jax import lax
from jax.experimental import pallas as pl
from jax.experimental.pallas import tpu as pltpu
from jax.experimental.pallas import tpu_sc as plsc

NC = 2
NS = 16
NW = NC * NS

CHUNK = 128
DH = 64
NB = 4
G = 8


def _sc_scatter_rows(feat, src, dst, n_pad):
    ngroups = src.shape[1]
    rows_per_tile = n_pad // NS
    assert ngroups % 2 == 0 and ngroups >= 4

    mesh = plsc.VectorSubcoreMesh(core_axis_name="c", subcore_axis_name="s")

    @functools.partial(
        pl.kernel,
        mesh=mesh,
        compiler_params=pltpu.CompilerParams(use_tc_tiling_on_sc=False),
        out_type=jax.ShapeDtypeStruct((NC, n_pad, DH), jnp.float32),
        scratch_types=[
            pltpu.VMEM((2, G, CHUNK), jnp.int32),
            pltpu.VMEM((2, G, CHUNK), jnp.int32),
            pltpu.VMEM((NB, CHUNK, DH), jnp.float32),
            pltpu.VMEM_SHARED((n_pad, DH), jnp.float32),
            pltpu.VMEM_SHARED((n_pad, DH), jnp.float32),
            [pltpu.SemaphoreType.DMA] * NB,
            [pltpu.SemaphoreType.DMA] * 2,
        ],
    )
    def k(feat_hbm, src_hbm, dst_hbm, out_hbm, srcg, dstg, rows_v,
          tab_sh, acc_sh, gsems, isems):
        c = lax.axis_index("c")
        s = lax.axis_index("s")
        row0 = s * rows_per_tile

        def stage_idx(g, p):
            pltpu.async_copy(src_hbm.at[s, g], srcg.at[p], isems[p])
            pltpu.async_copy(dst_hbm.at[s, g], dstg.at[p], isems[p])

        def idx_wait(p):
            for _ in range(2):
                pltpu.make_async_copy(src_hbm.at[s, 0], srcg.at[p],
                                      isems[p]).wait()

        stage_idx(0, 0)
        pltpu.sync_copy(
            feat_hbm.at[pl.ds(row0, rows_per_tile), pl.ds(c * DH, DH)],
            tab_sh.at[pl.ds(row0, rows_per_tile)])

        zblk = jnp.zeros((16,), jnp.float32)
        for r in range(8):
            for l in range(DH // 16):
                rows_v[0, r, pl.ds(l * 16, 16)] = zblk

        def zero_body(j, _):
            pltpu.sync_copy(rows_v.at[0, pl.ds(0, 8)],
                            acc_sh.at[pl.ds(row0 + j * 8, 8)])
            return 0

        lax.fori_loop(0, rows_per_tile // 8, zero_body, 0)
        idx_wait(0)
        plsc.subcore_barrier()

        def gather(p, jj, b):
            pltpu.async_copy(tab_sh.at[srcg.at[p, jj]], rows_v.at[b],
                             gsems[b])

        def gwait(b):
            pltpu.make_async_copy(feat_hbm.at[pl.ds(0, CHUNK), pl.ds(0, DH)],
                                  rows_v.at[b], gsems[b]).wait()

        def scatter(p, t, b):
            pltpu.sync_copy(rows_v.at[b], acc_sh.at[dstg.at[p, t]],
                            add=True)

        def run_group(g, p, stage_next, gather_next):
            if stage_next:
                stage_idx(g + 1, 1 - p)
            for t in range(G):
                b = t % NB
                if t == G - NB and gather_next:
                    idx_wait(1 - p)
                gwait(b)
                scatter(p, t, b)
                if t < G - NB:
                    gather(p, t + NB, b)
                elif gather_next:
                    gather(1 - p, t + NB - G, b)
            return

        for b in range(NB):
            gather(0, b, b)
        run_group(0, 0, stage_next=True, gather_next=True)

        def pair_body(i, _):
            ga = 1 + 2 * i
            run_group(ga, 1, stage_next=True, gather_next=True)
            run_group(ga + 1, 0, stage_next=True, gather_next=True)
            return 0

        lax.fori_loop(0, (ngroups - 2) // 2, pair_body, 0)

        run_group(ngroups - 1, 1, stage_next=False, gather_next=False)
        plsc.subcore_barrier()

        pltpu.sync_copy(acc_sh.at[pl.ds(row0, rows_per_tile)],
                        out_hbm.at[c, pl.ds(row0, rows_per_tile)])

    return k(feat, src, dst)


def _hidden_kernel(p_ref, x_ref, w_ref, b_ref, o_ref):
    agg = jnp.concatenate([p_ref[0], p_ref[1]], axis=1) + x_ref[...]
    h = jnp.dot(agg, w_ref[...], preferred_element_type=jnp.float32)
    o_ref[...] = jnp.maximum(h + b_ref[...], 0.0)


def _heads_kernel(p_ref, h_ref, w2_ref, w3_ref, mu_ref, lv_ref):
    agg = jnp.concatenate([p_ref[0], p_ref[1]], axis=1) + h_ref[...]
    mu_ref[...] = jnp.dot(agg, w2_ref[...], preferred_element_type=jnp.float32)
    lv_ref[...] = jnp.dot(agg, w3_ref[...], preferred_element_type=jnp.float32)


def _adj_kernel(a_ref, b_ref, o_ref):
    o_ref[...] = lax.dot_general(
        a_ref[...], b_ref[...], (((1,), (1,)), ((), ())),
        preferred_element_type=jnp.float32)


def kernel(x, edge_index, W1, b1, W2, W3):
    n, d_in = x.shape
    e = edge_index.shape[1]
    h2 = W2.shape[1]

    src = edge_index[0].astype(jnp.int32)
    dst = edge_index[1].astype(jnp.int32)

    n_pad = ((n + NS * 8 - 1) // (NS * 8)) * (NS * 8)
    estep = NS * CHUNK * G * 2
    e_pad = ((e + estep - 1) // estep) * estep
    if e_pad != e:
        pad = e_pad - e
        src = jnp.concatenate([src, jnp.zeros((pad,), jnp.int32)])
        dst = jnp.concatenate([dst, jnp.full((pad,), n_pad - 1, jnp.int32)])
    ngroups = e_pad // (NS * CHUNK * G)
    src = src.reshape(NS, ngroups, G, CHUNK)
    dst = dst.reshape(NS, ngroups, G, CHUNK)

    x_padded = jnp.concatenate(
        [x, jnp.zeros((n_pad - n, d_in), jnp.float32)], axis=0)

    parts1 = _sc_scatter_rows(x_padded, src, dst, n_pad)

    rb = 1000
    grid = (n // rb,)
    hidden1 = pl.pallas_call(
        _hidden_kernel,
        grid=grid,
        in_specs=[
            pl.BlockSpec((NC, rb, DH), lambda i: (0, i, 0)),
            pl.BlockSpec((rb, d_in), lambda i: (i, 0)),
            pl.BlockSpec((d_in, d_in), lambda i: (0, 0)),
            pl.BlockSpec((d_in,), lambda i: (0,)),
        ],
        out_specs=pl.BlockSpec((rb, d_in), lambda i: (i, 0)),
        out_shape=jax.ShapeDtypeStruct((n_pad, d_in), jnp.float32),
    )(parts1, x, W1, b1)

    parts2 = _sc_scatter_rows(hidden1, src, dst, n_pad)

    mu, logvar = pl.pallas_call(
        _heads_kernel,
        grid=grid,
        in_specs=[
            pl.BlockSpec((NC, rb, DH), lambda i: (0, i, 0)),
            pl.BlockSpec((rb, d_in), lambda i: (i, 0)),
            pl.BlockSpec((d_in, h2), lambda i: (0, 0)),
            pl.BlockSpec((d_in, h2), lambda i: (0, 0)),
        ],
        out_specs=[
            pl.BlockSpec((rb, h2), lambda i: (i, 0)),
            pl.BlockSpec((rb, h2), lambda i: (i, 0)),
        ],
        out_shape=[
            jax.ShapeDtypeStruct((n, h2), jnp.float32),
            jax.ShapeDtypeStruct((n, h2), jnp.float32),
        ],
    )(parts2, hidden1, W2, W3)

    arb, acb = 2048, 2560
    gi = (n + arb - 1) // arb
    gj = (n + acb - 1) // acb
    adj = pl.pallas_call(
        _adj_kernel,
        grid=(gi, gj),
        in_specs=[
            pl.BlockSpec((arb, h2), lambda i, j: (i, 0)),
            pl.BlockSpec((acb, h2), lambda i, j: (j, 0)),
        ],
        out_specs=pl.BlockSpec((arb, acb), lambda i, j: (i, j)),
        out_shape=jax.ShapeDtypeStruct((n, n), jnp.float32),
    )(mu, mu)

    return (adj, mu, logvar)
